# baseline (speedup 1.0000x reference)
import functools

import jax
import jax.numpy as jnp
from jax import lax
from jax.experimental import pallas as pl
from jax.experimental.pallas import tpu as pltpu
from jax.experimental.pallas import tpu_sc as plsc

D = 128
CHUNK = 128
NC = 2
NS = 16
NW = NC * NS
K = 2
QI = 4


def _agg_sc(node, adj_e, n_pad, tch, n0, n1):
    rpt = n_pad // NS
    nzc = rpt // CHUNK

    mesh = plsc.VectorSubcoreMesh(core_axis_name="c", subcore_axis_name="s")

    @functools.partial(
        pl.kernel,
        mesh=mesh,
        out_type=jax.ShapeDtypeStruct((NC * n_pad, D), jnp.float32),
        scratch_types=[
            [pltpu.VMEM((QI, CHUNK), jnp.int32)] * 2,
            [pltpu.VMEM((CHUNK, D), jnp.float32)] * K,
            pltpu.VMEM_SHARED((n_pad, D), jnp.float32),
            [pltpu.SemaphoreType.DMA] * QI,
            [pltpu.SemaphoreType.DMA] * QI,
            [pltpu.SemaphoreType.DMA] * K,
        ],
    )
    def agg(node_hbm, adj_hbm, out_hbm, sd_v, rows_v, acc,
            isems, dsems, gsems):
        c = lax.axis_index("c")
        s = lax.axis_index("s")
        sidx, didx = sd_v
        ncw = jnp.where(c == 0, n0, n1)
        o_w = jnp.where(c == 0, 0, NS * n0) + s * ncw
        my_cpw = jnp.maximum(0, jnp.minimum(ncw, tch - o_w))

        zero16 = jnp.zeros((16,), jnp.float32)

        def zbody(k, carry):
            i = k // (D // 16)
            j = k % (D // 16)
            rows_v[0][i, pl.ds(j * 16, 16)] = zero16
            return carry

        lax.fori_loop(0, CHUNK * (D // 16), zbody, 0)
        for k in range(nzc):
            pltpu.sync_copy(rows_v[0], acc.at[pl.ds(s * rpt + k * CHUNK, CHUNK)])
        plsc.subcore_barrier()

        def fire_idx(j, q):
            e0 = (o_w + j) * CHUNK
            pltpu.async_copy(adj_hbm.at[0, pl.ds(e0, CHUNK)], sidx.at[q], isems[q])
            pltpu.async_copy(adj_hbm.at[1, pl.ds(e0, CHUNK)], didx.at[q], dsems[q])

        def wait_idx(sems, q):
            pltpu.make_async_copy(
                adj_hbm.at[0, pl.ds(0, CHUNK)], sidx.at[q], sems[q]).wait()

        def fire_gather(q, b):
            pltpu.async_copy(node_hbm.at[sidx.at[q]], rows_v[b], gsems[b])

        for q in range(QI):
            @pl.when(q < my_cpw)
            def _():
                fire_idx(q, q)
        for b in range(K):
            @pl.when(b < my_cpw)
            def _():
                wait_idx(isems, b)
                fire_gather(b, b)

        def body(t, carry):
            j0 = t * QI
            for u in range(QI):
                j = j0 + u
                b = u % K
                pltpu.make_async_copy(
                    node_hbm.at[sidx.at[u]], rows_v[b], gsems[b]).wait()
                wait_idx(dsems, u)
                pltpu.sync_copy(rows_v[b], acc.at[didx.at[u]], add=True)

                @pl.when(j + QI < my_cpw)
                def _():
                    fire_idx(j + QI, u)

                @pl.when(j + K < my_cpw)
                def _():
                    wait_idx(isems, (u + K) % QI)
                    fire_gather((u + K) % QI, b)
            return carry

        lax.fori_loop(0, my_cpw // QI, body, 0)
        plsc.subcore_barrier()

        for k in range(nzc):
            row = s * rpt + k * CHUNK
            pltpu.sync_copy(acc.at[pl.ds(row, CHUNK)],
                            out_hbm.at[pl.ds(c * n_pad + row, CHUNK)])

    return agg(node, adj_e)


def _mlp_body(scale_ref, x_ref, p0_ref, p1_ref,
              w1_ref, b1_ref, g1_ref, be1_ref,
              w2_ref, b2_ref, g2_ref, be2_ref,
              w3_ref, b3_ref, gn_ref, bn_ref, out_ref):
    def ln_relu(h, g, b):
        mu = jnp.mean(h, axis=1, keepdims=True)
        xc = h - mu
        var = jnp.mean(xc * xc, axis=1, keepdims=True)
        return jnp.maximum(xc * lax.rsqrt(var + 1e-5) * g + b, 0.0)

    dn = (((1,), (1,)), ((), ()))
    bf = jnp.bfloat16
    h = scale_ref[0, 0] * x_ref[...] + p0_ref[...] + p1_ref[...]
    h = lax.dot_general(h.astype(bf), w1_ref[...].astype(bf), dn,
                        preferred_element_type=jnp.float32)
    h = ln_relu(h + b1_ref[...], g1_ref[...], be1_ref[...])
    h = lax.dot_general(h.astype(bf), w2_ref[...].astype(bf), dn,
                        preferred_element_type=jnp.float32)
    h = ln_relu(h + b2_ref[...], g2_ref[...], be2_ref[...])
    h = lax.dot_general(h.astype(bf), w3_ref[...].astype(bf), dn,
                        preferred_element_type=jnp.float32)
    out_ref[...] = ln_relu(h + b3_ref[...], gn_ref[...], bn_ref[...])


def kernel(node, adj, batch_ptr, eps, W1, b1, g1, be1, W2, b2, g2, be2,
           W3, b3, gn, bn):
    n, d = node.shape
    e = adj.shape[1]
    assert d == D

    tch = -(-e // CHUNK)
    assert tch % QI == 0
    n0 = QI * max(1, -(-tch // (2 * NS * QI)))
    n1 = n0

    adj_e = adj.astype(jnp.int32)
    if e % CHUNK:
        pad = tch * CHUNK - e
        adj_e = jnp.concatenate(
            [adj_e,
             jnp.stack([jnp.zeros((pad,), jnp.int32),
                        jnp.full((pad,), n, jnp.int32)])], axis=1)

    n_pad = -(-(n + 1) // (NS * CHUNK)) * NS * CHUNK
    parts = _agg_sc(node, adj_e, n_pad, tch, n0, n1)

    scale = (1.0 + eps).astype(jnp.float32).reshape(1, 1)

    br = 512
    nb = -(-n // br)
    assert n_pad % br == 0
    po = n_pad // br
    full = lambda shp: pl.BlockSpec(shp, lambda i: (0, 0))
    row_blk = pl.BlockSpec((br, D), lambda i: (i, 0))
    p0_blk = pl.BlockSpec((br, D), lambda i: (i, 0))
    p1_blk = pl.BlockSpec((br, D), lambda i: (po + i, 0))
    vec = lambda: full((1, D))

    out = pl.pallas_call(
        _mlp_body,
        grid=(nb,),
        in_specs=[
            full((1, 1)),
            row_blk,
            p0_blk, p1_blk,
            full((D, D)), vec(), vec(), vec(),
            full((D, D)), vec(), vec(), vec(),
            full((D, D)), vec(), vec(), vec(),
        ],
        out_specs=row_blk,
        out_shape=jax.ShapeDtypeStruct((n, D), jnp.float32),
    )(
        scale, node, parts, parts,
        W1, b1.reshape(1, D), g1.reshape(1, D), be1.reshape(1, D),
        W2, b2.reshape(1, D), g2.reshape(1, D), be2.reshape(1, D),
        W3, b3.reshape(1, D), gn.reshape(1, D), bn.reshape(1, D),
    )
    return out


# speedup vs baseline: 1.0713x; 1.0713x over previous
import functools

import jax
import jax.numpy as jnp
from jax import lax
from jax.experimental import pallas as pl
from jax.experimental.pallas import tpu as pltpu
from jax.experimental.pallas import tpu_sc as plsc

D = 128
CHUNK = 128
NC = 2
NS = 16
NW = NC * NS
K = 2
QI = 4


def _agg_sc(node, adj_e, n_pad, tch, n0, n1):
    rpt = n_pad // NS
    nzc = rpt // CHUNK

    mesh = plsc.VectorSubcoreMesh(core_axis_name="c", subcore_axis_name="s")

    @functools.partial(
        pl.kernel,
        mesh=mesh,
        out_type=jax.ShapeDtypeStruct((NC * n_pad, D), jnp.float32),
        scratch_types=[
            [pltpu.VMEM((QI, CHUNK), jnp.int32)] * 2,
            [pltpu.VMEM((CHUNK, D), jnp.float32)] * K,
            pltpu.VMEM_SHARED((n_pad, D), jnp.float32),
            [pltpu.SemaphoreType.DMA] * QI,
            [pltpu.SemaphoreType.DMA] * QI,
            [pltpu.SemaphoreType.DMA] * K,
        ],
    )
    def agg(node_hbm, adj_hbm, out_hbm, sd_v, rows_v, acc,
            isems, dsems, gsems):
        c = lax.axis_index("c")
        s = lax.axis_index("s")
        sidx, didx = sd_v
        ncw = jnp.where(c == 0, n0, n1)
        o_w = jnp.where(c == 0, 0, NS * n0) + s * ncw
        my_cpw = jnp.maximum(0, jnp.minimum(ncw, tch - o_w))

        zero16 = jnp.zeros((16,), jnp.float32)

        def zbody(k, carry):
            i = k // (D // 16)
            j = k % (D // 16)
            rows_v[0][i, pl.ds(j * 16, 16)] = zero16
            return carry

        lax.fori_loop(0, CHUNK * (D // 16), zbody, 0)
        for k in range(nzc):
            pltpu.sync_copy(rows_v[0], acc.at[pl.ds(s * rpt + k * CHUNK, CHUNK)])
        plsc.subcore_barrier()

        def fire_idx(j, q):
            e0 = (o_w + j) * CHUNK
            pltpu.async_copy(adj_hbm.at[0, pl.ds(e0, CHUNK)], sidx.at[q], isems[q])
            pltpu.async_copy(adj_hbm.at[1, pl.ds(e0, CHUNK)], didx.at[q], dsems[q])

        def wait_idx(sems, q):
            pltpu.make_async_copy(
                adj_hbm.at[0, pl.ds(0, CHUNK)], sidx.at[q], sems[q]).wait()

        def fire_gather(q, b):
            pltpu.async_copy(node_hbm.at[sidx.at[q]], rows_v[b], gsems[b])

        for q in range(QI):
            @pl.when(q < my_cpw)
            def _():
                fire_idx(q, q)
        for b in range(K):
            @pl.when(b < my_cpw)
            def _():
                wait_idx(isems, b)
                fire_gather(b, b)

        def body(t, carry):
            j0 = t * QI
            for u in range(QI):
                j = j0 + u
                b = u % K
                pltpu.make_async_copy(
                    node_hbm.at[sidx.at[u]], rows_v[b], gsems[b]).wait()
                wait_idx(dsems, u)
                pltpu.sync_copy(rows_v[b], acc.at[didx.at[u]], add=True)

                @pl.when(j + QI < my_cpw)
                def _():
                    fire_idx(j + QI, u)

                @pl.when(j + K < my_cpw)
                def _():
                    wait_idx(isems, (u + K) % QI)
                    fire_gather((u + K) % QI, b)
            return carry

        lax.fori_loop(0, my_cpw // QI, body, 0)
        plsc.subcore_barrier()

        for k in range(nzc):
            row = s * rpt + k * CHUNK
            pltpu.sync_copy(acc.at[pl.ds(row, CHUNK)],
                            out_hbm.at[pl.ds(c * n_pad + row, CHUNK)])

    return agg(node, adj_e)


def _mlp_body(scale_ref, x_ref, p0_ref, p1_ref,
              w1_ref, b1_ref, g1_ref, be1_ref,
              w2_ref, b2_ref, g2_ref, be2_ref,
              w3_ref, b3_ref, gn_ref, bn_ref, out_ref):
    def ln_relu(h, g, b):
        mu = jnp.mean(h, axis=1, keepdims=True)
        xc = h - mu
        var = jnp.mean(xc * xc, axis=1, keepdims=True)
        return jnp.maximum(xc * lax.rsqrt(var + 1e-5) * g + b, 0.0)

    dn = (((1,), (1,)), ((), ()))
    h = scale_ref[0, 0] * x_ref[...] + p0_ref[...] + p1_ref[...]
    h = lax.dot_general(h, w1_ref[...], dn, preferred_element_type=jnp.float32)
    h = ln_relu(h + b1_ref[...], g1_ref[...], be1_ref[...])
    h = lax.dot_general(h, w2_ref[...], dn, preferred_element_type=jnp.float32)
    h = ln_relu(h + b2_ref[...], g2_ref[...], be2_ref[...])
    h = lax.dot_general(h, w3_ref[...], dn, preferred_element_type=jnp.float32)
    out_ref[...] = ln_relu(h + b3_ref[...], gn_ref[...], bn_ref[...])


def kernel(node, adj, batch_ptr, eps, W1, b1, g1, be1, W2, b2, g2, be2,
           W3, b3, gn, bn):
    n, d = node.shape
    e = adj.shape[1]
    assert d == D

    tch = -(-e // CHUNK)
    assert tch % QI == 0
    n0 = QI * max(1, -(-tch // (2 * NS * QI)))
    n1 = n0

    adj_e = adj.astype(jnp.int32)
    if e % CHUNK:
        pad = tch * CHUNK - e
        adj_e = jnp.concatenate(
            [adj_e,
             jnp.stack([jnp.zeros((pad,), jnp.int32),
                        jnp.full((pad,), n, jnp.int32)])], axis=1)

    n_pad = -(-(n + 1) // (NS * CHUNK)) * NS * CHUNK
    parts = _agg_sc(node, adj_e, n_pad, tch, n0, n1)

    scale = (1.0 + eps).astype(jnp.float32).reshape(1, 1)

    br = 1024
    nb = -(-n // br)
    assert n_pad % br == 0
    po = n_pad // br
    full = lambda shp: pl.BlockSpec(shp, lambda i: (0, 0))
    row_blk = pl.BlockSpec((br, D), lambda i: (i, 0))
    p0_blk = pl.BlockSpec((br, D), lambda i: (i, 0))
    p1_blk = pl.BlockSpec((br, D), lambda i: (po + i, 0))
    vec = lambda: full((1, D))

    out = pl.pallas_call(
        _mlp_body,
        grid=(nb,),
        in_specs=[
            full((1, 1)),
            row_blk,
            p0_blk, p1_blk,
            full((D, D)), vec(), vec(), vec(),
            full((D, D)), vec(), vec(), vec(),
            full((D, D)), vec(), vec(), vec(),
        ],
        out_specs=row_blk,
        out_shape=jax.ShapeDtypeStruct((n, D), jnp.float32),
    )(
        scale, node, parts, parts,
        W1, b1.reshape(1, D), g1.reshape(1, D), be1.reshape(1, D),
        W2, b2.reshape(1, D), g2.reshape(1, D), be2.reshape(1, D),
        W3, b3.reshape(1, D), gn.reshape(1, D), bn.reshape(1, D),
    )
    return out
